# Initial kernel scaffold; baseline (speedup 1.0000x reference)
import jax
import jax.numpy as jnp
from jax.experimental import pallas as pl


def kernel(inputs, embeddings):
    raise NotImplementedError("write your pallas kernel here")


# speedup vs baseline: 5.2061x; 5.2061x over previous
import functools

import jax
import jax.numpy as jnp
from jax import lax
from jax.experimental import pallas as pl
from jax.experimental.pallas import tpu as pltpu
from jax.experimental.pallas import tpu_sc as plsc

N_POINTS = 1048576
EMB_DIM = 32
TABLE_ROWS = 291
TABLE_WORDS = TABLE_ROWS * EMB_DIM

SCALE = (32.0, 128.0, 128.0)
OFFSET = (0.0, 33.0, 162.0)
MAXV = (32.0, 161.0, 290.0)

NC, NS, L = 2, 16, 16
NW = NC * NS
PW = N_POINTS // NW
CHUNK = 512
N_CHUNKS = PW // CHUNK
GROUPS = CHUNK // L


def _body(inputs_hbm, table_hbm, out_hbm, in_v, t_v, out_v):
    wid = lax.axis_index("s") * NC + lax.axis_index("c")
    base_w = wid * PW

    pltpu.sync_copy(table_hbm, t_v)

    iota16 = lax.iota(jnp.int32, L)
    ocol = iota16 * 96

    def chunk_body(ch, carry):
        base = base_w + ch * CHUNK
        for a in range(3):
            pltpu.sync_copy(inputs_hbm.at[pl.ds(a * N_POINTS + base, CHUNK)],
                            in_v.at[pl.ds(a * CHUNK, CHUNK)])

        def group_body(g, carry2):
            obase = ocol + g * (L * 96)
            for a in range(3):
                u = in_v[pl.ds(a * CHUNK + g * L, L)]
                c = u * SCALE[a] + OFFSET[a]
                t = c.astype(jnp.int32).astype(jnp.float32)
                lo = lax.clamp(OFFSET[a], t, MAXV[a])
                hi = lax.clamp(OFFSET[a], t + 1.0, MAXV[a])
                wlo = 1.0 - jnp.abs(lo - c)
                whi = 1.0 - jnp.abs(hi - c)
                ilo = lo.astype(jnp.int32) * EMB_DIM
                ihi = hi.astype(jnp.int32) * EMB_DIM
                od = obase + a * EMB_DIM
                for d in range(EMB_DIM):
                    vlo = plsc.load_gather(t_v, [ilo + d])
                    vhi = plsc.load_gather(t_v, [ihi + d])
                    res = wlo * vlo + whi * vhi
                    plsc.store_scatter(out_v, [od + d], res)
            return carry2

        lax.fori_loop(0, GROUPS, group_body, 0, unroll=False)
        pltpu.sync_copy(out_v, out_hbm.at[pl.ds(base * 96, CHUNK * 96)])
        return carry

    lax.fori_loop(0, N_CHUNKS, chunk_body, 0, unroll=False)


@jax.jit
def kernel(inputs, embeddings):
    mesh = plsc.VectorSubcoreMesh(core_axis_name="c", subcore_axis_name="s")
    k = pl.kernel(
        _body,
        out_type=jax.ShapeDtypeStruct((N_POINTS * 96,), jnp.float32),
        mesh=mesh,
        compiler_params=pltpu.CompilerParams(needs_layout_passes=False),
        scratch_types=[
            pltpu.VMEM((3 * CHUNK,), jnp.float32),
            pltpu.VMEM((TABLE_WORDS,), jnp.float32),
            pltpu.VMEM((CHUNK * 96,), jnp.float32),
        ],
    )
    inputs_t = inputs.T.reshape(-1)
    table = embeddings.reshape(-1)
    out = k(inputs_t, table)
    return out.reshape(N_POINTS, 96)


# speedup vs baseline: 20.9323x; 4.0207x over previous
import functools

import jax
import jax.numpy as jnp
from jax import lax
from jax.experimental import pallas as pl
from jax.experimental.pallas import tpu as pltpu
from jax.experimental.pallas import tpu_sc as plsc

N_POINTS = 1048576
EMB_DIM = 32
TABLE_ROWS = 291
TABLE_WORDS = TABLE_ROWS * EMB_DIM

SCALE = (32.0, 128.0, 128.0)
OFFSET = (0.0, 33.0, 162.0)

NC, NS, L = 2, 16, 16
NW = NC * NS
PW = N_POINTS // NW
CHUNK = 512
N_CHUNKS = PW // CHUNK
GROUPS = CHUNK // L
D_VREGS = (TABLE_WORDS - EMB_DIM) // L


def _body(inputs_hbm, table_hbm, out_hbm, in_v, t_v, d_v, lo_v, fr_v, out_v):
    wid = lax.axis_index("s") * NC + lax.axis_index("c")
    base_w = wid * PW

    pltpu.sync_copy(table_hbm, t_v)

    def diff_body(i, carry):
        a = t_v[pl.ds(i * L, L)]
        b = t_v[pl.ds(i * L + EMB_DIM, L)]
        d_v[pl.ds(i * L, L)] = b - a
        return carry

    lax.fori_loop(0, D_VREGS, diff_body, 0, unroll=False)

    def chunk_body(ch, carry):
        base = base_w + ch * CHUNK
        for a in range(3):
            pltpu.sync_copy(inputs_hbm.at[pl.ds(a * N_POINTS + base, CHUNK)],
                            in_v.at[pl.ds(a * CHUNK, CHUNK)])

        def group_body(g, carry2):
            for a in range(3):
                u = in_v[pl.ds(a * CHUNK + g * L, L)]
                c = u * SCALE[a] + OFFSET[a]
                li = c.astype(jnp.int32)
                fr = c - li.astype(jnp.float32)
                lo_v[pl.ds(a * CHUNK + g * L, L)] = li * EMB_DIM
                fr_v[pl.ds(a * CHUNK + g * L, L)] = fr
            return carry2

        lax.fori_loop(0, GROUPS, group_body, 0, unroll=False)

        def pgroup_body(g, carry2):
            lov = [lo_v[pl.ds(a * CHUNK + g * L, L)] for a in range(3)]
            frv = [fr_v[pl.ds(a * CHUNK + g * L, L)] for a in range(3)]
            for i in range(L):
                ob = (g * L + i) * 96
                for a in range(3):
                    row = lov[a][i]
                    fr = jnp.full((L,), frv[a][i], dtype=jnp.float32)
                    t0 = t_v[pl.ds(row, L)]
                    t1 = t_v[pl.ds(row + L, L)]
                    d0 = d_v[pl.ds(row, L)]
                    d1 = d_v[pl.ds(row + L, L)]
                    out_v[pl.ds(ob + a * EMB_DIM, L)] = t0 + fr * d0
                    out_v[pl.ds(ob + a * EMB_DIM + L, L)] = t1 + fr * d1
            return carry2

        lax.fori_loop(0, GROUPS, pgroup_body, 0, unroll=False)

        pltpu.sync_copy(out_v, out_hbm.at[pl.ds(base * 96, CHUNK * 96)])
        return carry

    lax.fori_loop(0, N_CHUNKS, chunk_body, 0, unroll=False)


@jax.jit
def kernel(inputs, embeddings):
    mesh = plsc.VectorSubcoreMesh(core_axis_name="c", subcore_axis_name="s")
    k = pl.kernel(
        _body,
        out_type=jax.ShapeDtypeStruct((N_POINTS * 96,), jnp.float32),
        mesh=mesh,
        compiler_params=pltpu.CompilerParams(needs_layout_passes=False),
        scratch_types=[
            pltpu.VMEM((3 * CHUNK,), jnp.float32),
            pltpu.VMEM((TABLE_WORDS,), jnp.float32),
            pltpu.VMEM((TABLE_WORDS,), jnp.float32),
            pltpu.VMEM((3 * CHUNK,), jnp.int32),
            pltpu.VMEM((3 * CHUNK,), jnp.float32),
            pltpu.VMEM((CHUNK * 96,), jnp.float32),
        ],
    )
    inputs_t = inputs.T.reshape(-1)
    table = embeddings.reshape(-1)
    out = k(inputs_t, table)
    return out.reshape(N_POINTS, 96)
